# baseline (speedup 1.0000x reference)
import functools

import jax
import jax.numpy as jnp
from jax import lax
from jax.experimental import pallas as pl
from jax.experimental.pallas import tpu as pltpu
from jax.experimental.pallas import tpu_sc as plsc

B = 4096
L = 200
D = 64
LANES = 16
NC = 2
NS = 16
NW = NC * NS
SEQ_PER_W = B // NW
CH_SEQ = 2
NCHUNK = SEQ_PER_W // CH_SEQ
NBUF = 4


def _emb_body(idx_hbm, pos_hbm, tok_hbm, out_hbm, pos_v,
              idx0, idx1, idx2, idx3,
              rows0, rows1, rows2, rows3,
              gs0, gs1, gs2, gs3, ss0, ss1, ss2, ss3, is0, is1, is2, is3):
    wid = lax.axis_index("s") * NC + lax.axis_index("c")
    w_seq = wid * SEQ_PER_W

    idx_b = (idx0, idx1, idx2, idx3)
    rows_b = (rows0, rows1, rows2, rows3)
    gsem = (gs0, gs1, gs2, gs3)
    ssem = (ss0, ss1, ss2, ss3)
    isem = (is0, is1, is2, is3)

    def fire_idx(c, b):
        pltpu.async_copy(
            idx_hbm.at[pl.ds(w_seq + c * CH_SEQ, CH_SEQ)], idx_b[b], isem[b])

    def wait_idx(b):
        pltpu.make_async_copy(
            idx_hbm.at[pl.ds(0, CH_SEQ)], idx_b[b], isem[b]).wait()

    def fire_gather(b):
        for k in range(CH_SEQ):
            pltpu.async_copy(
                tok_hbm.at[idx_b[b].at[k]], rows_b[b].at[k], gsem[b])

    def wait_gather(b):
        for k in range(CH_SEQ):
            pltpu.make_async_copy(
                tok_hbm.at[idx_b[b].at[k]], rows_b[b].at[k], gsem[b]).wait()

    def fire_store(c, b):
        pltpu.async_copy(
            rows_b[b], out_hbm.at[pl.ds(w_seq + c * CH_SEQ, CH_SEQ)], ssem[b])

    def wait_store(b):
        pltpu.make_async_copy(
            rows_b[b], out_hbm.at[pl.ds(0, CH_SEQ)], ssem[b]).wait()

    pltpu.sync_copy(pos_hbm, pos_v)

    fire_idx(0, 0)
    fire_idx(1, 1)
    fire_idx(2, 2)
    wait_idx(0)
    fire_gather(0)
    wait_idx(1)
    fire_gather(1)

    def outer(t, _):
        for b in range(NBUF):
            c = t * NBUF + b
            wait_gather(b)
            rows = rows_b[b]

            @plsc.parallel_loop(0, L, unroll=2)
            def _add(i):
                for j in range(D // LANES):
                    sl = pl.ds(j * LANES, LANES)
                    p = pos_v[i, sl]
                    for k in range(CH_SEQ):
                        rows[k, i, sl] = rows[k, i, sl] + p

            fire_store(c, b)

            bn = (b + 2) % NBUF

            @pl.when(c + 2 < NCHUNK)
            def _():
                @pl.when(c >= 2)
                def _():
                    wait_store(bn)
                wait_idx(bn)
                fire_gather(bn)

            @pl.when(c + 3 < NCHUNK)
            def _():
                fire_idx(c + 3, (b + 3) % NBUF)
        return 0

    lax.fori_loop(0, NCHUNK // NBUF, outer, 0, unroll=False)

    for b in range(NBUF):
        wait_store(b)


@jax.jit
def _emb(idx, position_table, token_table):
    mesh = plsc.VectorSubcoreMesh(core_axis_name="c", subcore_axis_name="s")
    return pl.kernel(
        _emb_body,
        mesh=mesh,
        compiler_params=pltpu.CompilerParams(use_tc_tiling_on_sc=False),
        out_type=jax.ShapeDtypeStruct((B, L, D), jnp.float32),
        scratch_types=[
            pltpu.VMEM((L, D), jnp.float32),
            pltpu.VMEM((CH_SEQ, L), jnp.int32),
            pltpu.VMEM((CH_SEQ, L), jnp.int32),
            pltpu.VMEM((CH_SEQ, L), jnp.int32),
            pltpu.VMEM((CH_SEQ, L), jnp.int32),
            pltpu.VMEM((CH_SEQ, L, D), jnp.float32),
            pltpu.VMEM((CH_SEQ, L, D), jnp.float32),
            pltpu.VMEM((CH_SEQ, L, D), jnp.float32),
            pltpu.VMEM((CH_SEQ, L, D), jnp.float32),
        ] + [pltpu.SemaphoreType.DMA] * 12,
    )(idx, position_table, token_table)


def kernel(inputs, token_table, position_table):
    idx = jnp.asarray(inputs, jnp.int32)
    return _emb(idx, position_table, token_table)


# speedup vs baseline: 1.0918x; 1.0918x over previous
import functools

import jax
import jax.numpy as jnp
from jax import lax
from jax.experimental import pallas as pl
from jax.experimental.pallas import tpu as pltpu
from jax.experimental.pallas import tpu_sc as plsc

B = 4096
L = 200
D = 64
DP = 128
LANES = 16
NC = 2
NS = 16
NW = NC * NS
SEQ_PER_W = B // NW
RPW = SEQ_PER_W * L
CHUNK = 40
NCH = RPW // CHUNK
QPS = L // CHUNK
NSLOT = 8


def _emb_body(idx_hbm, pos_hbm, tok_hbm, out_hbm, pos_v, idx_all,
              g0, g1, g2, g3, g4, g5, g6, g7,
              s0, s1, s2, s3, s4, s5, s6, s7,
              gs0, gs1, gs2, gs3, gs4, gs5, gs6, gs7,
              ss0, ss1, ss2, ss3, ss4, ss5, ss6, ss7, isem):
    wid = lax.axis_index("s") * NC + lax.axis_index("c")
    w_row = wid * RPW
    w_seq = wid * SEQ_PER_W

    grows = (g0, g1, g2, g3, g4, g5, g6, g7)
    srows = (s0, s1, s2, s3, s4, s5, s6, s7)
    gsem = (gs0, gs1, gs2, gs3, gs4, gs5, gs6, gs7)
    ssem = (ss0, ss1, ss2, ss3, ss4, ss5, ss6, ss7)

    def fire_gather(c, b):
        pltpu.async_copy(
            tok_hbm.at[idx_all.at[pl.ds(c * CHUNK, CHUNK)]], grows[b], gsem[b])

    def wait_gather(c, b):
        pltpu.make_async_copy(
            tok_hbm.at[idx_all.at[pl.ds(c * CHUNK, CHUNK)]],
            grows[b], gsem[b]).wait()

    def fire_store(seq, q, b):
        pltpu.async_copy(
            srows[b], out_hbm.at[seq, pl.ds(q * CHUNK, CHUNK)], ssem[b])

    def wait_store(b):
        pltpu.make_async_copy(
            srows[b], out_hbm.at[0, pl.ds(0, CHUNK)], ssem[b]).wait()

    pltpu.async_copy(idx_hbm.at[pl.ds(w_row, RPW)], idx_all, isem)
    pltpu.sync_copy(pos_hbm, pos_v)
    pltpu.make_async_copy(idx_hbm.at[pl.ds(0, RPW)], idx_all, isem).wait()

    for c in range(NSLOT // 2):
        fire_gather(c, c)

    def outer(t, _):
        for b in range(NSLOT):
            c = t * NSLOT + b
            wait_gather(c, b)
            seq = w_seq + c // QPS
            q = lax.rem(c, QPS)
            gr = grows[b]
            sr = srows[b]
            pbase = q * (CHUNK * D)

            @plsc.parallel_loop(0, CHUNK, unroll=2)
            def _add(i):
                for j in range(D // LANES):
                    sl = pl.ds(j * LANES, LANES)
                    p = pos_v[pl.ds(pbase + i * D + j * LANES, LANES)]
                    sr[i, sl] = gr[i, sl] + p

            fire_store(seq, q, b)

            bn = (b + 4) % NSLOT

            @pl.when(c + 4 < NCH)
            def _():
                @pl.when(c >= 4)
                def _():
                    wait_store(bn)
                fire_gather(c + 4, bn)
        return 0

    lax.fori_loop(0, NCH // NSLOT, outer, 0, unroll=False)

    for b in range(NSLOT):
        wait_store(b)


@jax.jit
def _emb(idx_flat, pos_flat, tok_pad):
    mesh = plsc.VectorSubcoreMesh(core_axis_name="c", subcore_axis_name="s")
    return pl.kernel(
        _emb_body,
        mesh=mesh,
        out_type=jax.ShapeDtypeStruct((B, L, D), jnp.float32),
        scratch_types=[
            pltpu.VMEM((L * D,), jnp.float32),
            pltpu.VMEM((RPW,), jnp.int32),
        ]
        + [pltpu.VMEM((CHUNK, DP), jnp.float32)] * NSLOT
        + [pltpu.VMEM((CHUNK, D), jnp.float32)] * NSLOT
        + [pltpu.SemaphoreType.DMA] * (2 * NSLOT + 1),
    )(idx_flat, pos_flat, tok_pad)


def kernel(inputs, token_table, position_table):
    idx_flat = jnp.asarray(inputs, jnp.int32).reshape(B * L)
    pos_flat = position_table.reshape(L * D)
    tok_pad = jnp.pad(token_table, ((0, 0), (0, DP - D)))
    return _emb(idx_flat, pos_flat, tok_pad)
